# Initial kernel scaffold; baseline (speedup 1.0000x reference)
import jax
import jax.numpy as jnp
from jax.experimental import pallas as pl


def kernel(x, batch, edge_index, params):
    raise NotImplementedError("write your pallas kernel here")


# speedup vs baseline: 2.7892x; 2.7892x over previous
import functools

import jax
import jax.numpy as jnp
from jax import lax
from jax.experimental import pallas as pl
from jax.experimental.pallas import tpu as pltpu
from jax.experimental.pallas import tpu_sc as plsc

F32 = jnp.float32
_HI = lax.Precision.HIGHEST
_EPS = 1e-5


def _dot_t(a, b):
    return lax.dot_general(a, b, (((1,), (1,)), ((), ())),
                           preferred_element_type=F32, precision=_HI)


def _dot_t_bf16(a, b):
    return lax.dot_general(a.astype(jnp.bfloat16), b.astype(jnp.bfloat16),
                           (((1,), (1,)), ((), ())),
                           preferred_element_type=F32)


def _bn0_body(x_ref, g_ref, b_ref, o_ref):
    x = x_ref[...]
    m = jnp.mean(x, axis=0, keepdims=True)
    v = jnp.mean((x - m) ** 2, axis=0, keepdims=True)
    o_ref[...] = (x - m) / jnp.sqrt(v + _EPS) * g_ref[...] + b_ref[...]


def _bn0(x, g, b):
    return pl.pallas_call(
        _bn0_body,
        out_shape=jax.ShapeDtypeStruct(x.shape, F32),
    )(x, g.reshape(1, -1), b.reshape(1, -1))


def _knn_body(n_valid, kk, hi_ref, hf_ref, idx_ref):
    hi = hi_ref[...]
    hf = hf_ref[...]
    r, np_ = hi.shape[0], hf.shape[0]
    d2i = jnp.sum(hi * hi, axis=1, keepdims=True)
    d2f = _dot_t(jnp.ones((1, hf.shape[1]), F32), hf * hf)
    dist = d2i + d2f - 2.0 * _dot_t_bf16(hi, hf)
    cols = lax.broadcasted_iota(jnp.int32, (r, np_), 1)
    inf = jnp.float32(jnp.inf)
    dist = jnp.where(cols < n_valid, dist, inf)
    outs = []
    for _ in range(kk):
        m = jnp.min(dist, axis=1, keepdims=True)
        cand = jnp.where(dist == m, cols, jnp.int32(2**30))
        j = jnp.min(cand, axis=1, keepdims=True)
        outs.append(j)
        dist = jnp.where(cols == j, inf, dist)
    idx_ref[...] = jnp.concatenate(outs, axis=1)


def _knn(hp, n_valid, kk):
    np_, dp = hp.shape
    r = 256
    return pl.pallas_call(
        functools.partial(_knn_body, n_valid, kk),
        grid=(np_ // r,),
        in_specs=[pl.BlockSpec((r, dp), lambda i: (i, 0)),
                  pl.BlockSpec((np_, dp), lambda i: (0, 0))],
        out_specs=pl.BlockSpec((r, kk), lambda i: (i, 0)),
        out_shape=jax.ShapeDtypeStruct((np_, kk), jnp.int32),
    )(hp, hp)


def _pq_body(h_ref, a_ref, b_ref, bias_ref, p_ref, q_ref):
    h = h_ref[...]
    p_ref[...] = _dot_t_bf16(h, a_ref[...]) + bias_ref[...]
    q_ref[...] = _dot_t_bf16(h, b_ref[...])


def _pq(hp, a, b, bias):
    np_, dp = hp.shape
    w = a.shape[0]
    blk = 512
    return pl.pallas_call(
        _pq_body,
        grid=(np_ // blk,),
        in_specs=[pl.BlockSpec((blk, dp), lambda i: (i, 0)),
                  pl.BlockSpec(a.shape, lambda i: (0, 0)),
                  pl.BlockSpec(b.shape, lambda i: (0, 0)),
                  pl.BlockSpec((1, w), lambda i: (0, 0))],
        out_specs=[pl.BlockSpec((blk, w), lambda i: (i, 0)),
                   pl.BlockSpec((blk, w), lambda i: (i, 0))],
        out_shape=[jax.ShapeDtypeStruct((np_, w), F32),
                   jax.ShapeDtypeStruct((np_, w), F32)],
    )(hp, a, b, bias.reshape(1, -1))


def _sc_gather(table, idx):
    bsz = idx.shape[0]
    w = table.shape[1]
    wp = ((w + 127) // 128) * 128
    if wp != w:
        table = jnp.zeros((table.shape[0], wp), F32).at[:, :w].set(table)
    info = plsc.get_sparse_core_info()
    nw = info.num_cores * info.num_subcores
    bpw = bsz // nw
    nch = bpw // 128
    mesh = plsc.VectorSubcoreMesh(core_axis_name="c", subcore_axis_name="s")

    @functools.partial(
        pl.kernel, mesh=mesh,
        out_type=jax.ShapeDtypeStruct((bsz, wp), F32),
        scratch_types=[pltpu.VMEM((128,), jnp.int32),
                       pltpu.VMEM((128, wp), F32),
                       pltpu.SemaphoreType.DMA],
    )
    def k(table_hbm, idx_hbm, out_hbm, idx_v, rows_v, sem):
        wid = lax.axis_index("s") * info.num_cores + lax.axis_index("c")
        base = wid * bpw

        def body(c, carry):
            off = base + c * 128
            pltpu.sync_copy(idx_hbm.at[pl.ds(off, 128)], idx_v)
            pltpu.async_copy(table_hbm.at[idx_v], rows_v, sem).wait()
            pltpu.sync_copy(rows_v, out_hbm.at[pl.ds(off, 128)])
            return carry

        lax.fori_loop(0, nch, body, 0)

    out = k(table, idx)
    return out[:, :w] if wp != w else out


def _stats_update(i, s_ref, y, e_valid):
    blk, w = y.shape
    rows = i * blk + lax.broadcasted_iota(jnp.int32, (blk, 1), 0)
    ym = jnp.where(rows < e_valid, y, 0.0)
    s0 = jnp.sum(ym, axis=0, keepdims=True)
    s1 = jnp.sum(ym * ym, axis=0, keepdims=True)
    upd = jnp.concatenate([s0, s1, jnp.zeros((6, w), F32)], axis=0)

    @pl.when(i == 0)
    def _():
        s_ref[...] = jnp.zeros_like(s_ref)

    s_ref[...] += upd


def _norm(y, st, vs, g, be, cnt):
    m = st[0:1, :] / cnt
    v = vs[0:1, :] / cnt
    return jnp.maximum((y - m) * (g / jnp.sqrt(v + _EPS)) + be, 0.0)


def _vsum_body(e_valid, cnt, y_ref, s_ref, v_ref):
    i = pl.program_id(0)
    y = y_ref[...]
    blk, w = y.shape
    m = s_ref[0:1, :] / cnt
    rows = i * blk + lax.broadcasted_iota(jnp.int32, (blk, 1), 0)
    d = jnp.where(rows < e_valid, y - m, 0.0)
    upd = jnp.concatenate([jnp.sum(d * d, axis=0, keepdims=True),
                           jnp.zeros((7, w), F32)], axis=0)

    @pl.when(i == 0)
    def _():
        v_ref[...] = jnp.zeros_like(v_ref)

    v_ref[...] += upd


def _vsum(y, st, e_valid, cnt):
    epad, w = y.shape
    blk = 1024
    return pl.pallas_call(
        functools.partial(_vsum_body, e_valid, cnt),
        grid=(epad // blk,),
        in_specs=[pl.BlockSpec((blk, w), lambda i: (i, 0)),
                  pl.BlockSpec((8, w), lambda i: (0, 0))],
        out_specs=pl.BlockSpec((8, w), lambda i: (0, 0)),
        out_shape=jax.ShapeDtypeStruct((8, w), F32),
    )(y, st)


def _conv_l0_body(e_valid, d, hi_ref, g_ref, w_ref, b_ref, y_ref, s_ref):
    i = pl.program_id(0)
    hi = hi_ref[...]
    xj = g_ref[...]
    bn, dp = hi.shape
    blk = xj.shape[0]
    xi = jnp.broadcast_to(hi[:, None, :], (bn, blk // bn, dp)).reshape(blk, dp)
    msg = jnp.concatenate(
        [xi[:, :d], xj[:, :d] - xi[:, :d],
         jnp.zeros((blk, 2 * dp - 2 * d), F32)], axis=1)
    y = _dot_t_bf16(msg, w_ref[...]) + b_ref[...]
    y_ref[...] = y
    _stats_update(i, s_ref, y, e_valid)


def _conv_l0(hp, g, w0p, bias, e_valid, d):
    epad, dp = g.shape
    w = w0p.shape[0]
    blk = 1024
    bn = blk // 16
    return pl.pallas_call(
        functools.partial(_conv_l0_body, e_valid, d),
        grid=(epad // blk,),
        in_specs=[pl.BlockSpec((bn, dp), lambda i: (i, 0)),
                  pl.BlockSpec((blk, dp), lambda i: (i, 0)),
                  pl.BlockSpec(w0p.shape, lambda i: (0, 0)),
                  pl.BlockSpec((1, w), lambda i: (0, 0))],
        out_specs=[pl.BlockSpec((blk, w), lambda i: (i, 0)),
                   pl.BlockSpec((8, w), lambda i: (0, 0))],
        out_shape=[jax.ShapeDtypeStruct((epad, w), F32),
                   jax.ShapeDtypeStruct((8, w), F32)],
    )(hp, g, w0p, bias.reshape(1, -1))


def _out_pre_body(e_valid, gr_ref, gc_ref, y_ref, s_ref):
    i = pl.program_id(0)
    y = gr_ref[...] + gc_ref[...]
    y_ref[...] = y
    _stats_update(i, s_ref, y, e_valid)


def _out_pre(gr, gc, e_valid):
    epad, w = gr.shape
    blk = 1024
    return pl.pallas_call(
        functools.partial(_out_pre_body, e_valid),
        grid=(epad // blk,),
        in_specs=[pl.BlockSpec((blk, w), lambda i: (i, 0)),
                  pl.BlockSpec((blk, w), lambda i: (i, 0))],
        out_specs=[pl.BlockSpec((blk, w), lambda i: (i, 0)),
                   pl.BlockSpec((8, w), lambda i: (0, 0))],
        out_shape=[jax.ShapeDtypeStruct((epad, w), F32),
                   jax.ShapeDtypeStruct((8, w), F32)],
    )(gr, gc)


def _layer_body(e_valid, cnt, y_ref, s_ref, vs_ref, g_ref, be_ref, w_ref,
                b2_ref, o_ref, s2_ref):
    i = pl.program_id(0)
    h = _norm(y_ref[...], s_ref[...], vs_ref[...], g_ref[...], be_ref[...],
              cnt)
    y2 = _dot_t_bf16(h, w_ref[...]) + b2_ref[...]
    o_ref[...] = y2
    _stats_update(i, s2_ref, y2, e_valid)


def _layer(y, st, vs, g, be, w2, b2, e_valid, cnt):
    epad, win = y.shape
    wout = w2.shape[0]
    blk = 1024
    return pl.pallas_call(
        functools.partial(_layer_body, e_valid, cnt),
        grid=(epad // blk,),
        in_specs=[pl.BlockSpec((blk, win), lambda i: (i, 0)),
                  pl.BlockSpec((8, win), lambda i: (0, 0)),
                  pl.BlockSpec((8, win), lambda i: (0, 0)),
                  pl.BlockSpec((1, win), lambda i: (0, 0)),
                  pl.BlockSpec((1, win), lambda i: (0, 0)),
                  pl.BlockSpec((wout, win), lambda i: (0, 0)),
                  pl.BlockSpec((1, wout), lambda i: (0, 0))],
        out_specs=[pl.BlockSpec((blk, wout), lambda i: (i, 0)),
                   pl.BlockSpec((8, wout), lambda i: (0, 0))],
        out_shape=[jax.ShapeDtypeStruct((epad, wout), F32),
                   jax.ShapeDtypeStruct((8, wout), F32)],
    )(y, st, vs, g.reshape(1, -1), be.reshape(1, -1), w2,
      b2.reshape(1, -1))


def _conv_fin_body(cnt, y_ref, s_ref, vs_ref, g_ref, be_ref, x2_ref):
    h = _norm(y_ref[...], s_ref[...], vs_ref[...], g_ref[...], be_ref[...],
              cnt)
    blk = h.shape[0]
    bn = blk // 16
    r = lax.broadcasted_iota(jnp.int32, (bn, blk), 0)
    c = lax.broadcasted_iota(jnp.int32, (bn, blk), 1)
    sel = (c // 16 == r).astype(F32)
    x2_ref[...] = lax.dot_general(sel, h, (((1,), (0,)), ((), ())),
                                  preferred_element_type=F32,
                                  precision=_HI) * (1.0 / 16.0)


def _conv_fin(y, st, vs, g, be, cnt):
    epad, w = y.shape
    blk = 1024
    bn = blk // 16
    return pl.pallas_call(
        functools.partial(_conv_fin_body, cnt),
        grid=(epad // blk,),
        in_specs=[pl.BlockSpec((blk, w), lambda i: (i, 0)),
                  pl.BlockSpec((8, w), lambda i: (0, 0)),
                  pl.BlockSpec((8, w), lambda i: (0, 0)),
                  pl.BlockSpec((1, w), lambda i: (0, 0)),
                  pl.BlockSpec((1, w), lambda i: (0, 0))],
        out_specs=pl.BlockSpec((bn, w), lambda i: (i, 0)),
        out_shape=jax.ShapeDtypeStruct((epad // 16, w), F32),
    )(y, st, vs, g.reshape(1, -1), be.reshape(1, -1))


def _out_last_body(cnt, y_ref, s_ref, vs_ref, g_ref, be_ref, w_ref, b3_ref,
                   o_ref):
    h = _norm(y_ref[...], s_ref[...], vs_ref[...], g_ref[...], be_ref[...],
              cnt)
    o_ref[...] = _dot_t_bf16(h, w_ref[...]) + b3_ref[...]


def _out_last(y, st, vs, g, be, w3, b3, cnt):
    epad, win = y.shape
    blk = 1024
    w3p = jnp.zeros((128, win), F32).at[0:1, :].set(w3)
    b3p = jnp.broadcast_to(b3.reshape(1, 1), (1, 128)).astype(F32)
    return pl.pallas_call(
        functools.partial(_out_last_body, cnt),
        grid=(epad // blk,),
        in_specs=[pl.BlockSpec((blk, win), lambda i: (i, 0)),
                  pl.BlockSpec((8, win), lambda i: (0, 0)),
                  pl.BlockSpec((8, win), lambda i: (0, 0)),
                  pl.BlockSpec((1, win), lambda i: (0, 0)),
                  pl.BlockSpec((1, win), lambda i: (0, 0)),
                  pl.BlockSpec((128, win), lambda i: (0, 0)),
                  pl.BlockSpec((1, 128), lambda i: (0, 0))],
        out_specs=pl.BlockSpec((blk, 128), lambda i: (i, 0)),
        out_shape=jax.ShapeDtypeStruct((epad, 128), F32),
    )(y, st, vs, g.reshape(1, -1), be.reshape(1, -1), w3p, b3p)


def _pad_w(w, dp):
    wo, di = w.shape
    return jnp.zeros((wo, dp), F32).at[:, :di].set(w)


def _conv_block(hp, dcur, layers, n, kk, e, epad):
    w = layers[0]["W"].shape[0]
    dp = hp.shape[1]
    idx = _knn(hp, n, kk)
    idxf = idx[:n].reshape(-1)
    idxf = jnp.concatenate(
        [idxf, jnp.zeros((epad - e,), jnp.int32)])
    w0 = layers[0]["W"]
    w0p = jnp.zeros((w, 2 * dp), F32).at[:, :2 * dcur].set(w0)
    g = _sc_gather(hp, idxf)
    cnt = float(e)
    y1, s1 = _conv_l0(hp, g, w0p, layers[0]["b"], e, dcur)
    vs1 = _vsum(y1, s1, e, cnt)
    y2, s2 = _layer(y1, s1, vs1, layers[0]["g"], layers[0]["beta"],
                    layers[1]["W"], layers[1]["b"], e, cnt)
    vs2 = _vsum(y2, s2, e, cnt)
    y3, s3 = _layer(y2, s2, vs2, layers[1]["g"], layers[1]["beta"],
                    layers[2]["W"], layers[2]["b"], e, cnt)
    vs3 = _vsum(y3, s3, e, cnt)
    return _conv_fin(y3, s3, vs3, layers[2]["g"], layers[2]["beta"], cnt)


def kernel(x, batch, edge_index, params):
    n = x.shape[0]
    kk = 16
    e = n * kk
    npad = ((n + 255) // 256) * 256
    epad = npad * kk
    eedge = edge_index.shape[1]

    x1 = _bn0(x, params["bn0_g"], params["bn0_b"])

    hp = jnp.zeros((npad, 128), F32).at[:n, :4].set(x)
    x2 = _conv_block(hp, 4, params["mlp1"], n, kk, e, epad)

    hp = jnp.zeros((npad, 128), F32).at[:n, :4].set(x1).at[:n, 4:36].set(
        x2[:n])
    x2 = _conv_block(hp, 36, params["mlp2"], n, kk, e, epad)

    hp = jnp.zeros((npad, 128), F32).at[:n, :4].set(x1).at[:n, 4:68].set(
        x2[:n])
    x2 = _conv_block(hp, 68, params["mlp3"], n, kk, e, epad)

    hn = jnp.zeros((npad, 256), F32).at[:n, :4].set(x1).at[:n, 4:132].set(
        x2[:n])
    out_layers = params["out"]
    w0 = out_layers[0]["W"]
    din = w0.shape[1] // 2
    wr = _pad_w(w0[:, :din], 256)
    wc = _pad_w(w0[:, din:], 256)
    rt, ct = _pq(hn, wr, wc, out_layers[0]["b"])

    epad2 = ((eedge + 4095) // 4096) * 4096
    rowi = jnp.concatenate(
        [edge_index[0], jnp.zeros((epad2 - eedge,), jnp.int32)])
    coli = jnp.concatenate(
        [edge_index[1], jnp.zeros((epad2 - eedge,), jnp.int32)])
    gr = _sc_gather(rt, rowi)
    gc = _sc_gather(ct, coli)

    cnt = float(eedge)
    y1, s1 = _out_pre(gr, gc, eedge)
    vs1 = _vsum(y1, s1, eedge, cnt)
    y2, s2 = _layer(y1, s1, vs1, out_layers[0]["g"], out_layers[0]["beta"],
                    out_layers[1]["W"], out_layers[1]["b"], eedge, cnt)
    vs2 = _vsum(y2, s2, eedge, cnt)
    yf = _out_last(y2, s2, vs2, out_layers[1]["g"], out_layers[1]["beta"],
                   out_layers[2]["W"], out_layers[2]["b"], cnt)
    return yf[:eedge, :1]
